# Initial kernel scaffold; baseline (speedup 1.0000x reference)
import jax
import jax.numpy as jnp
from jax.experimental import pallas as pl


def kernel(x, edge_index, edge_attr, hidden, node_W1, node_b1, node_W2, node_b2, node_g, node_beta, edge_W1, edge_b1, edge_W2, edge_b2, edge_g, edge_beta):
    raise NotImplementedError("write your pallas kernel here")


# speedup vs baseline: 3.0114x; 3.0114x over previous
import jax
import jax.numpy as jnp
from jax import lax
from jax.experimental import pallas as pl
from jax.experimental.pallas import tpu as pltpu
from jax.experimental.pallas import tpu_sc as plsc

N = 10000
E = 160000
C = 256
P = 2

NC = 2
NS = 16
NW = NC * NS

K = 40
EPT = E // NW
NCH = EPT // K
EPS = E // NS
NCHS = EPS // K
NP2 = 10240
NPS = NP2 // NS
CH = C // 2


def _slots(p, fn):
    @pl.when(p == 0)
    def _():
        fn(0)

    @pl.when(p == 1)
    def _():
        fn(1)


def _gather_body(xa, xb, dst3, src3, ga, gb, idxd, idxs, rows_a, rows_b,
                 sga, sgb, ssa, ssb):
    cid = lax.axis_index("c")
    sid = lax.axis_index("s")
    wid = sid * NC + cid
    base = wid * EPT
    pltpu.sync_copy(dst3.at[wid], idxd)
    pltpu.sync_copy(src3.at[wid], idxs)
    pltpu.async_copy(xa.at[idxd.at[0]], rows_a.at[0], sga.at[0])
    pltpu.async_copy(xb.at[idxs.at[0]], rows_b.at[0], sgb.at[0])

    def body(ch, carry):
        p = lax.rem(ch, 2)

        @pl.when(ch + 1 < NCH)
        def _():
            q = lax.rem(ch + 1, 2)

            def start_next(qs):
                @pl.when(ch >= 1)
                def _():
                    pltpu.make_async_copy(
                        rows_a.at[qs], ga.at[pl.ds(0, K)], ssa.at[qs]).wait()
                    pltpu.make_async_copy(
                        rows_b.at[qs], gb.at[pl.ds(0, K)], ssb.at[qs]).wait()
                pltpu.async_copy(xa.at[idxd.at[ch + 1]], rows_a.at[qs], sga.at[qs])
                pltpu.async_copy(xb.at[idxs.at[ch + 1]], rows_b.at[qs], sgb.at[qs])

            _slots(q, start_next)

        def fin(ps):
            pltpu.make_async_copy(
                xa.at[idxd.at[ch]], rows_a.at[ps], sga.at[ps]).wait()
            pltpu.make_async_copy(
                xb.at[idxs.at[ch]], rows_b.at[ps], sgb.at[ps]).wait()
            pltpu.async_copy(rows_a.at[ps], ga.at[pl.ds(base + ch * K, K)], ssa.at[ps])
            pltpu.async_copy(rows_b.at[ps], gb.at[pl.ds(base + ch * K, K)], ssb.at[ps])

        _slots(p, fin)
        return carry

    lax.fori_loop(0, NCH, body, 0)
    for q in (0, 1):
        pltpu.make_async_copy(rows_a.at[q], ga.at[pl.ds(0, K)], ssa.at[q]).wait()
        pltpu.make_async_copy(rows_b.at[q], gb.at[pl.ds(0, K)], ssb.at[q]).wait()


def _gather_sc(xa, xb, dst3, src3):
    mesh = plsc.VectorSubcoreMesh(core_axis_name="c", subcore_axis_name="s")
    f = pl.kernel(
        _gather_body,
        out_type=[jax.ShapeDtypeStruct((E, C), jnp.float32),
                  jax.ShapeDtypeStruct((E, C), jnp.float32)],
        mesh=mesh,
        scratch_types=[
            pltpu.VMEM((NCH, K), jnp.int32),
            pltpu.VMEM((NCH, K), jnp.int32),
            pltpu.VMEM((2, K, C), jnp.float32),
            pltpu.VMEM((2, K, C), jnp.float32),
            pltpu.SemaphoreType.DMA((2,)),
            pltpu.SemaphoreType.DMA((2,)),
            pltpu.SemaphoreType.DMA((2,)),
            pltpu.SemaphoreType.DMA((2,)),
        ],
    )
    return f(xa, xb, dst3, src3)


def _scatter_body(ue, dst3s, zi, agg, idxb, rows, spmem, sld, ssc):
    cid = lax.axis_index("c")
    sid = lax.axis_index("s")
    base = sid * EPS
    slab = pl.ds(sid * NPS, NPS)

    pltpu.sync_copy(dst3s.at[sid], idxb)
    pltpu.sync_copy(zi.at[slab], spmem.at[slab])
    plsc.subcore_barrier()

    def run(cs):
        col = cs * CH
        pltpu.async_copy(
            ue.at[pl.ds(base, K), pl.ds(col, CH)], rows.at[0], sld.at[0])

        def body(ch, carry):
            p = lax.rem(ch, 2)

            @pl.when(ch + 1 < NCHS)
            def _():
                q = lax.rem(ch + 1, 2)

                def start_next(qs):
                    @pl.when(ch >= 1)
                    def _():
                        pltpu.make_async_copy(
                            rows.at[qs], spmem.at[idxb.at[0]], ssc.at[qs]).wait()
                    pltpu.async_copy(
                        ue.at[pl.ds(base + (ch + 1) * K, K), pl.ds(col, CH)],
                        rows.at[qs], sld.at[qs])

                _slots(q, start_next)

            def fin(ps):
                pltpu.make_async_copy(
                    ue.at[pl.ds(base, K), pl.ds(col, CH)],
                    rows.at[ps], sld.at[ps]).wait()
                pltpu.async_copy(
                    rows.at[ps], spmem.at[idxb.at[ch]], ssc.at[ps], add=True)

            _slots(p, fin)
            return carry

        lax.fori_loop(0, NCHS, body, 0)
        for q in (0, 1):
            pltpu.make_async_copy(
                rows.at[q], spmem.at[idxb.at[0]], ssc.at[q]).wait()
        plsc.subcore_barrier()
        pltpu.sync_copy(spmem.at[slab], agg.at[cs].at[slab])

    _slots(cid, run)


def _scatter_sc(ue, dst3s, zi):
    mesh = plsc.VectorSubcoreMesh(core_axis_name="c", subcore_axis_name="s")
    f = pl.kernel(
        _scatter_body,
        out_type=jax.ShapeDtypeStruct((NC, NP2, CH), jnp.float32),
        mesh=mesh,
        scratch_types=[
            pltpu.VMEM((NCHS, K), jnp.int32),
            pltpu.VMEM((2, K, CH), jnp.float32),
            pltpu.VMEM_SHARED((NP2, CH), jnp.float32),
            pltpu.SemaphoreType.DMA((2,)),
            pltpu.SemaphoreType.DMA((2,)),
        ],
    )
    return f(ue, dst3s, zi)


def _proj_tc(x_ref, wa, wb, xa_ref, xb_ref):
    xa_ref[...] = jnp.dot(x_ref[...], wa[...], preferred_element_type=jnp.float32)
    xb_ref[...] = jnp.dot(x_ref[...], wb[...], preferred_element_type=jnp.float32)


def _proj_call(x, wa, wb):
    BN = 1000
    return pl.pallas_call(
        _proj_tc,
        grid=(N // BN,),
        in_specs=[pl.BlockSpec((BN, C), lambda i: (i, 0)),
                  pl.BlockSpec((C, C), lambda i: (0, 0)),
                  pl.BlockSpec((C, C), lambda i: (0, 0))],
        out_specs=[pl.BlockSpec((BN, C), lambda i: (i, 0)),
                   pl.BlockSpec((BN, C), lambda i: (i, 0))],
        out_shape=[jax.ShapeDtypeStruct((N, C), jnp.float32),
                   jax.ShapeDtypeStruct((N, C), jnp.float32)],
    )(x, wa, wb)


def _ln_tail(u, g, beta):
    m = jnp.mean(u, axis=-1, keepdims=True)
    v = jnp.mean((u - m) ** 2, axis=-1, keepdims=True)
    return (u - m) * lax.rsqrt(v + 1e-5) * g + beta


def _edge_tc(ga_ref, gb_ref, ea_ref, hd_ref, w1c, w1d, b1, w2, b2, g, beta,
             out_ref):
    pre = (ga_ref[...] + gb_ref[...] + b1[...]
           + jnp.dot(ea_ref[...], w1c[...], preferred_element_type=jnp.float32)
           + jnp.dot(hd_ref[...], w1d[...], preferred_element_type=jnp.float32))
    h = jnp.maximum(pre, 0.0)
    u = jnp.dot(h, w2[...], preferred_element_type=jnp.float32) + b2[...]
    out_ref[...] = ea_ref[...] + _ln_tail(u, g[...], beta[...])


def _edge_call(ga, gb, ea, hd, w1c, w1d, b1, w2, b2, g, beta):
    BE = 1000
    dspec = pl.BlockSpec((BE, C), lambda i: (i, 0))
    wspec = pl.BlockSpec((C, C), lambda i: (0, 0))
    vspec = pl.BlockSpec((1, C), lambda i: (0, 0))
    return pl.pallas_call(
        _edge_tc,
        grid=(E // BE,),
        in_specs=[dspec, dspec, dspec, dspec, wspec, wspec, vspec, wspec,
                  vspec, vspec, vspec],
        out_specs=dspec,
        out_shape=jax.ShapeDtypeStruct((E, C), jnp.float32),
    )(ga, gb, ea, hd, w1c, w1d, b1, w2, b2, g, beta)


def _node_tc(x_ref, al_ref, ah_ref, w1a, w1bl, w1bh, b1, w2, b2, g, beta,
             out_ref):
    pre = (b1[...]
           + jnp.dot(x_ref[...], w1a[...], preferred_element_type=jnp.float32)
           + jnp.dot(al_ref[...], w1bl[...], preferred_element_type=jnp.float32)
           + jnp.dot(ah_ref[...], w1bh[...], preferred_element_type=jnp.float32))
    h = jnp.maximum(pre, 0.0)
    u = jnp.dot(h, w2[...], preferred_element_type=jnp.float32) + b2[...]
    out_ref[...] = x_ref[...] + _ln_tail(u, g[...], beta[...])


def _node_call(x, al, ah, w1a, w1bl, w1bh, b1, w2, b2, g, beta):
    BN = 1000
    dspec = pl.BlockSpec((BN, C), lambda i: (i, 0))
    hspec = pl.BlockSpec((BN, CH), lambda i: (i, 0))
    wspec = pl.BlockSpec((C, C), lambda i: (0, 0))
    w2spec = pl.BlockSpec((CH, C), lambda i: (0, 0))
    vspec = pl.BlockSpec((1, C), lambda i: (0, 0))
    return pl.pallas_call(
        _node_tc,
        grid=(N // BN,),
        in_specs=[dspec, hspec, hspec, wspec, w2spec, w2spec, vspec, wspec,
                  vspec, vspec, vspec],
        out_specs=dspec,
        out_shape=jax.ShapeDtypeStruct((N, C), jnp.float32),
    )(x, al, ah, w1a, w1bl, w1bh, b1, w2, b2, g, beta)


def kernel(x, edge_index, edge_attr, hidden, node_W1, node_b1, node_W2,
           node_b2, node_g, node_beta, edge_W1, edge_b1, edge_W2, edge_b2,
           edge_g, edge_beta):
    src = edge_index[0]
    dst = edge_index[1]
    dst3 = dst.reshape(NW, NCH, K)
    src3 = src.reshape(NW, NCH, K)
    dst3s = dst.reshape(NS, NCHS, K)
    zi = jnp.zeros((NP2, CH), jnp.float32)

    for i in range(P):
        ew1 = edge_W1[i]
        xa, xb = _proj_call(x, ew1[:C], ew1[C:2 * C])
        ga, gb = _gather_sc(xa, xb, dst3, src3)
        ue = _edge_call(
            ga, gb, edge_attr, hidden, ew1[2 * C:3 * C], ew1[3 * C:],
            edge_b1[i].reshape(1, C), edge_W2[i], edge_b2[i].reshape(1, C),
            edge_g[i].reshape(1, C), edge_beta[i].reshape(1, C))
        agg2 = _scatter_sc(ue, dst3s, zi)
        nw1 = node_W1[i]
        x = _node_call(
            x, agg2[0, :N], agg2[1, :N], nw1[:C], nw1[C:C + CH], nw1[C + CH:],
            node_b1[i].reshape(1, C), node_W2[i], node_b2[i].reshape(1, C),
            node_g[i].reshape(1, C), node_beta[i].reshape(1, C))
        edge_attr = ue
    return (x, edge_attr)
